# baseline (speedup 1.0000x reference)
import functools

import jax
import jax.numpy as jnp
from jax import lax
from jax.experimental import pallas as pl
from jax.experimental.pallas import tpu as pltpu
from jax.experimental.pallas import tpu_sc as plsc

N = 10000
D_IN = 128
D_H = 64
E = 320000

NC, NS = 2, 16
NW = NC * NS
NPAD = 10240
ROWS_PER_TILE = NPAD // NS
CHUNK = 128
CPW = 80
EPW = CPW * CHUNK
EPAD = EPW * NW
CCOL = D_H

BLK = 512


CH = CPW // 2


def _sc_agg_body(with_mark, z_hbm, src_hbm, dst_hbm, acc_out,
                 acc_sp, sidx, didx, rows_a, rows_b, gsem_a, gsem_b):
    c = lax.axis_index("c")
    s = lax.axis_index("s")

    zeros16 = jnp.zeros((16,), jnp.float32)
    onehot16 = jnp.where(lax.iota(jnp.int32, 16) == 0, 1.0, 0.0)

    def _zb(i, carry):
        for k in range(128 // 16):
            rows_a[i, pl.ds(16 * k, 16)] = zeros16
        return carry
    lax.fori_loop(0, CHUNK, _zb, 0)

    r0 = pl.multiple_of(s * ROWS_PER_TILE, ROWS_PER_TILE)
    for j in range(ROWS_PER_TILE // CHUNK):
        pltpu.sync_copy(rows_a, acc_sp.at[pl.ds(r0 + j * CHUNK, CHUNK)])

    plsc.subcore_barrier()

    wid = s * NC + c
    cb = pl.multiple_of(wid * CPW, 8)

    for h in range(CPW // CH):
        hb = pl.multiple_of(cb + h * CH, 8)
        pltpu.sync_copy(src_hbm.at[pl.ds(hb, CH)], sidx)
        pltpu.sync_copy(dst_hbm.at[pl.ds(hb, CH)], didx)
        pltpu.async_copy(z_hbm.at[sidx.at[0]], rows_a, gsem_a)
        pltpu.async_copy(z_hbm.at[sidx.at[1]], rows_b, gsem_b)

        def _mark(rows):
            if with_mark:
                for r in range(CHUNK):
                    rows[r, pl.ds(CCOL, 16)] = onehot16

        def _edges(j, carry):
            ca = 2 * j
            last = j >= CH // 2 - 1
            pltpu.make_async_copy(z_hbm.at[sidx.at[ca]], rows_a, gsem_a).wait()
            _mark(rows_a)
            pltpu.sync_copy(rows_a, acc_sp.at[didx.at[ca]], add=True)

            @pl.when(jnp.logical_not(last))
            def _():
                pltpu.async_copy(z_hbm.at[sidx.at[ca + 2]], rows_a, gsem_a)
            pltpu.make_async_copy(z_hbm.at[sidx.at[ca + 1]], rows_b,
                                  gsem_b).wait()
            _mark(rows_b)
            pltpu.sync_copy(rows_b, acc_sp.at[didx.at[ca + 1]], add=True)

            @pl.when(jnp.logical_not(last))
            def _():
                pltpu.async_copy(z_hbm.at[sidx.at[ca + 3]], rows_b, gsem_b)
            return carry
        lax.fori_loop(0, CH // 2, _edges, 0)

    plsc.subcore_barrier()

    o0 = pl.multiple_of(c * NPAD + r0, ROWS_PER_TILE)
    pltpu.sync_copy(acc_sp.at[pl.ds(r0, ROWS_PER_TILE)],
                    acc_out.at[pl.ds(o0, ROWS_PER_TILE)])


def _make_sc_agg(with_mark):
    return pl.kernel(
        functools.partial(_sc_agg_body, with_mark),
        out_type=jax.ShapeDtypeStruct((NC * NPAD, 128), jnp.float32),
        mesh=plsc.VectorSubcoreMesh(core_axis_name="c", subcore_axis_name="s"),
        scratch_types=[
            pltpu.VMEM_SHARED((NPAD, 128), jnp.float32),
            pltpu.VMEM((CH, CHUNK), jnp.int32),
            pltpu.VMEM((CH, CHUNK), jnp.int32),
            pltpu.VMEM((CHUNK, 128), jnp.float32),
            pltpu.VMEM((CHUNK, 128), jnp.float32),
            pltpu.SemaphoreType.DMA,
            pltpu.SemaphoreType.DMA,
        ],
    )


_sc_agg_cnt = _make_sc_agg(True)
_sc_agg = _make_sc_agg(False)


def _tc1_body(x_ref, wl_ref, wr_ref, zxr_ref):
    xb = x_ref[...]
    dn = (((1,), (1,)), ((), ()))
    z = lax.dot_general(xb, wl_ref[...], dn, preferred_element_type=jnp.float32)
    xr = lax.dot_general(xb, wr_ref[...], dn, preferred_element_type=jnp.float32)
    zxr_ref[...] = jnp.concatenate([z, xr], axis=1)


_tc1 = pl.pallas_call(
    _tc1_body,
    grid=(NPAD // BLK,),
    in_specs=[
        pl.BlockSpec((BLK, D_IN), lambda i: (i, 0)),
        pl.BlockSpec((D_H, D_IN), lambda i: (0, 0)),
        pl.BlockSpec((D_H, D_IN), lambda i: (0, 0)),
    ],
    out_specs=pl.BlockSpec((BLK, 2 * D_H), lambda i: (i, 0)),
    out_shape=jax.ShapeDtypeStruct((NPAD, 2 * D_H), jnp.float32),
)


def _tc2_body(parts_ref, b_ref, zxr1_ref, wl_ref, wr_ref,
              h_ref, cnt_ref, zxr2_ref):
    ssum = parts_ref[0, :, :D_H] + parts_ref[1, :, :D_H]
    cnt = jnp.maximum(parts_ref[0, :, CCOL] + parts_ref[1, :, CCOL], 1.0)
    xr = zxr1_ref[:, D_H:]
    h = jnp.maximum(ssum / cnt[:, None] + b_ref[...] + xr, 0.0)
    h_ref[...] = h
    cnt_ref[...] = cnt
    dn = (((1,), (1,)), ((), ()))
    z2 = lax.dot_general(h, wl_ref[...], dn, preferred_element_type=jnp.float32)
    xr2 = lax.dot_general(h, wr_ref[...], dn, preferred_element_type=jnp.float32)
    zxr2_ref[...] = jnp.concatenate([z2, xr2], axis=1)


_tc2 = pl.pallas_call(
    _tc2_body,
    grid=(NPAD // BLK,),
    in_specs=[
        pl.BlockSpec((NC, BLK, 128), lambda i: (0, i, 0)),
        pl.BlockSpec((1, D_H), lambda i: (0, 0)),
        pl.BlockSpec((BLK, 2 * D_H), lambda i: (i, 0)),
        pl.BlockSpec((D_H, D_H), lambda i: (0, 0)),
        pl.BlockSpec((D_H, D_H), lambda i: (0, 0)),
    ],
    out_specs=[
        pl.BlockSpec((BLK, D_H), lambda i: (i, 0)),
        pl.BlockSpec((BLK,), lambda i: (i,)),
        pl.BlockSpec((BLK, 2 * D_H), lambda i: (i, 0)),
    ],
    out_shape=[
        jax.ShapeDtypeStruct((NPAD, D_H), jnp.float32),
        jax.ShapeDtypeStruct((NPAD,), jnp.float32),
        jax.ShapeDtypeStruct((NPAD, 2 * D_H), jnp.float32),
    ],
)


def _tc3_body(parts_ref, cnt_in_ref, b_ref, zxr2_ref, wlin_ref, blin_ref,
              h_ref, out_ref):
    ssum = parts_ref[0, :, :D_H] + parts_ref[1, :, :D_H]
    cnt = cnt_in_ref[...]
    xr = zxr2_ref[:, D_H:]
    h = jnp.maximum(ssum / cnt[:, None] + b_ref[...] + xr, 0.0)
    h_ref[...] = h
    out_ref[...] = jnp.sum(h * wlin_ref[...], axis=1) + blin_ref[0, 0]


_tc3 = pl.pallas_call(
    _tc3_body,
    grid=(NPAD // BLK,),
    in_specs=[
        pl.BlockSpec((NC, BLK, 128), lambda i: (0, i, 0)),
        pl.BlockSpec((BLK,), lambda i: (i,)),
        pl.BlockSpec((1, D_H), lambda i: (0, 0)),
        pl.BlockSpec((BLK, 2 * D_H), lambda i: (i, 0)),
        pl.BlockSpec((1, D_H), lambda i: (0, 0)),
        pl.BlockSpec((1, 1), lambda i: (0, 0)),
    ],
    out_specs=[
        pl.BlockSpec((BLK, D_H), lambda i: (i, 0)),
        pl.BlockSpec((BLK,), lambda i: (i,)),
    ],
    out_shape=[
        jax.ShapeDtypeStruct((NPAD, D_H), jnp.float32),
        jax.ShapeDtypeStruct((NPAD,), jnp.float32),
    ],
)


def kernel(x, edge_index, W1l, b1l, W1r, W2l, b2l, W2r, Wlin, blin):
    x_p = jnp.pad(x, ((0, NPAD - N), (0, 0)))
    pad_len = EPAD - E
    pad_idx = (N + jnp.arange(pad_len, dtype=jnp.int32) % (NPAD - N))
    src_p = jnp.concatenate([edge_index[0], pad_idx]).reshape(EPAD // CHUNK,
                                                              CHUNK)
    dst_p = jnp.concatenate([edge_index[1], pad_idx]).reshape(EPAD // CHUNK,
                                                              CHUNK)

    zxr1 = _tc1(x_p, W1l, W1r)
    parts1 = _sc_agg_cnt(zxr1, src_p, dst_p).reshape(NC, NPAD, 128)
    h1, cnt1, zxr2 = _tc2(parts1, b1l.reshape(1, D_H), zxr1, W2l, W2r)
    parts2 = _sc_agg(zxr2, src_p, dst_p).reshape(NC, NPAD, 128)
    h2, out = _tc3(parts2, cnt1, b2l.reshape(1, D_H), zxr2, Wlin,
                   blin.reshape(1, 1))
    return (out[:N], h2[:N])


# speedup vs baseline: 1.0149x; 1.0149x over previous
import functools

import jax
import jax.numpy as jnp
from jax import lax
from jax.experimental import pallas as pl
from jax.experimental.pallas import tpu as pltpu
from jax.experimental.pallas import tpu_sc as plsc

N = 10000
D_IN = 128
D_H = 64
E = 320000

NC, NS = 2, 16
NW = NC * NS
NPAD = 10240
ROWS_PER_TILE = NPAD // NS
CHUNK = 128
CPW = 80
EPW = CPW * CHUNK
EPAD = EPW * NW
CCOL = D_H

BLK = 512


CH = CPW // 2


def _sc_agg_body(with_mark, z_hbm, src_hbm, dst_hbm, acc_out,
                 acc_sp, sidx, didx, rows_a, rows_b, gsem_a, gsem_b):
    c = lax.axis_index("c")
    s = lax.axis_index("s")

    zeros16 = jnp.zeros((16,), jnp.float32)
    onehot16 = jnp.where(lax.iota(jnp.int32, 16) == 0, 1.0, 0.0)

    def _zb(i, carry):
        for k in range(128 // 16):
            rows_b[i, pl.ds(16 * k, 16)] = zeros16
        return carry
    lax.fori_loop(0, CHUNK, _zb, 0)

    wid = s * NC + c
    cb = pl.multiple_of(wid * CPW, 8)
    pltpu.sync_copy(src_hbm.at[pl.ds(cb, CH)], sidx)
    pltpu.sync_copy(dst_hbm.at[pl.ds(cb, CH)], didx)
    pltpu.async_copy(z_hbm.at[sidx.at[0]], rows_a, gsem_a)

    r0 = pl.multiple_of(s * ROWS_PER_TILE, ROWS_PER_TILE)
    for j in range(ROWS_PER_TILE // CHUNK):
        pltpu.sync_copy(rows_b, acc_sp.at[pl.ds(r0 + j * CHUNK, CHUNK)])

    plsc.subcore_barrier()

    for h in range(CPW // CH):
        if h > 0:
            hb = pl.multiple_of(cb + h * CH, 8)
            pltpu.sync_copy(src_hbm.at[pl.ds(hb, CH)], sidx)
            pltpu.sync_copy(dst_hbm.at[pl.ds(hb, CH)], didx)
            pltpu.async_copy(z_hbm.at[sidx.at[0]], rows_a, gsem_a)
        pltpu.async_copy(z_hbm.at[sidx.at[1]], rows_b, gsem_b)

        def _mark(rows):
            if with_mark:
                for r in range(CHUNK):
                    rows[r, pl.ds(CCOL, 16)] = onehot16

        def _edges(j, carry):
            ca = 2 * j
            last = j >= CH // 2 - 1
            pltpu.make_async_copy(z_hbm.at[sidx.at[ca]], rows_a, gsem_a).wait()
            _mark(rows_a)
            pltpu.sync_copy(rows_a, acc_sp.at[didx.at[ca]], add=True)

            @pl.when(jnp.logical_not(last))
            def _():
                pltpu.async_copy(z_hbm.at[sidx.at[ca + 2]], rows_a, gsem_a)
            pltpu.make_async_copy(z_hbm.at[sidx.at[ca + 1]], rows_b,
                                  gsem_b).wait()
            _mark(rows_b)
            pltpu.sync_copy(rows_b, acc_sp.at[didx.at[ca + 1]], add=True)

            @pl.when(jnp.logical_not(last))
            def _():
                pltpu.async_copy(z_hbm.at[sidx.at[ca + 3]], rows_b, gsem_b)
            return carry
        lax.fori_loop(0, CH // 2, _edges, 0)

    plsc.subcore_barrier()

    o0 = pl.multiple_of(c * NPAD + r0, ROWS_PER_TILE)
    pltpu.sync_copy(acc_sp.at[pl.ds(r0, ROWS_PER_TILE)],
                    acc_out.at[pl.ds(o0, ROWS_PER_TILE)])


def _make_sc_agg(with_mark):
    return pl.kernel(
        functools.partial(_sc_agg_body, with_mark),
        out_type=jax.ShapeDtypeStruct((NC * NPAD, 128), jnp.float32),
        mesh=plsc.VectorSubcoreMesh(core_axis_name="c", subcore_axis_name="s"),
        scratch_types=[
            pltpu.VMEM_SHARED((NPAD, 128), jnp.float32),
            pltpu.VMEM((CH, CHUNK), jnp.int32),
            pltpu.VMEM((CH, CHUNK), jnp.int32),
            pltpu.VMEM((CHUNK, 128), jnp.float32),
            pltpu.VMEM((CHUNK, 128), jnp.float32),
            pltpu.SemaphoreType.DMA,
            pltpu.SemaphoreType.DMA,
        ],
    )


_sc_agg_cnt = _make_sc_agg(True)
_sc_agg = _make_sc_agg(False)


def _tc1_body(x_ref, wl_ref, wr_ref, zxr_ref):
    xb = x_ref[...]
    dn = (((1,), (1,)), ((), ()))
    z = lax.dot_general(xb, wl_ref[...], dn, preferred_element_type=jnp.float32)
    xr = lax.dot_general(xb, wr_ref[...], dn, preferred_element_type=jnp.float32)
    zxr_ref[...] = jnp.concatenate([z, xr], axis=1)


_tc1 = pl.pallas_call(
    _tc1_body,
    grid=(NPAD // BLK,),
    in_specs=[
        pl.BlockSpec((BLK, D_IN), lambda i: (i, 0)),
        pl.BlockSpec((D_H, D_IN), lambda i: (0, 0)),
        pl.BlockSpec((D_H, D_IN), lambda i: (0, 0)),
    ],
    out_specs=pl.BlockSpec((BLK, 2 * D_H), lambda i: (i, 0)),
    out_shape=jax.ShapeDtypeStruct((NPAD, 2 * D_H), jnp.float32),
)


def _tc2_body(parts_ref, b_ref, zxr1_ref, wl_ref, wr_ref,
              cnt_ref, zxr2_ref):
    ssum = parts_ref[0, :, :D_H] + parts_ref[1, :, :D_H]
    cnt = jnp.maximum(parts_ref[0, :, CCOL] + parts_ref[1, :, CCOL], 1.0)
    xr = zxr1_ref[:, D_H:]
    h = jnp.maximum(ssum / cnt[:, None] + b_ref[...] + xr, 0.0)
    cnt_ref[...] = cnt
    dn = (((1,), (1,)), ((), ()))
    z2 = lax.dot_general(h, wl_ref[...], dn, preferred_element_type=jnp.float32)
    xr2 = lax.dot_general(h, wr_ref[...], dn, preferred_element_type=jnp.float32)
    zxr2_ref[...] = jnp.concatenate([z2, xr2], axis=1)


_tc2 = pl.pallas_call(
    _tc2_body,
    grid=(NPAD // BLK,),
    in_specs=[
        pl.BlockSpec((NC, BLK, 128), lambda i: (0, i, 0)),
        pl.BlockSpec((1, D_H), lambda i: (0, 0)),
        pl.BlockSpec((BLK, 2 * D_H), lambda i: (i, 0)),
        pl.BlockSpec((D_H, D_H), lambda i: (0, 0)),
        pl.BlockSpec((D_H, D_H), lambda i: (0, 0)),
    ],
    out_specs=[
        pl.BlockSpec((BLK,), lambda i: (i,)),
        pl.BlockSpec((BLK, 2 * D_H), lambda i: (i, 0)),
    ],
    out_shape=[
        jax.ShapeDtypeStruct((NPAD,), jnp.float32),
        jax.ShapeDtypeStruct((NPAD, 2 * D_H), jnp.float32),
    ],
)


def _tc3_body(parts_ref, cnt_in_ref, b_ref, zxr2_ref, wlin_ref, blin_ref,
              h_ref, out_ref):
    ssum = parts_ref[0, :, :D_H] + parts_ref[1, :, :D_H]
    cnt = cnt_in_ref[...]
    xr = zxr2_ref[:, D_H:]
    h = jnp.maximum(ssum / cnt[:, None] + b_ref[...] + xr, 0.0)
    h_ref[...] = h
    out_ref[...] = jnp.sum(h * wlin_ref[...], axis=1) + blin_ref[0, 0]


_tc3 = pl.pallas_call(
    _tc3_body,
    grid=(NPAD // BLK,),
    in_specs=[
        pl.BlockSpec((NC, BLK, 128), lambda i: (0, i, 0)),
        pl.BlockSpec((BLK,), lambda i: (i,)),
        pl.BlockSpec((1, D_H), lambda i: (0, 0)),
        pl.BlockSpec((BLK, 2 * D_H), lambda i: (i, 0)),
        pl.BlockSpec((1, D_H), lambda i: (0, 0)),
        pl.BlockSpec((1, 1), lambda i: (0, 0)),
    ],
    out_specs=[
        pl.BlockSpec((BLK, D_H), lambda i: (i, 0)),
        pl.BlockSpec((BLK,), lambda i: (i,)),
    ],
    out_shape=[
        jax.ShapeDtypeStruct((NPAD, D_H), jnp.float32),
        jax.ShapeDtypeStruct((NPAD,), jnp.float32),
    ],
)


def kernel(x, edge_index, W1l, b1l, W1r, W2l, b2l, W2r, Wlin, blin):
    x_p = jnp.pad(x, ((0, NPAD - N), (0, 0)))
    pad_len = EPAD - E
    pad_idx = (N + jnp.arange(pad_len, dtype=jnp.int32) % (NPAD - N))
    src_p = jnp.concatenate([edge_index[0], pad_idx]).reshape(EPAD // CHUNK,
                                                              CHUNK)
    dst_p = jnp.concatenate([edge_index[1], pad_idx]).reshape(EPAD // CHUNK,
                                                              CHUNK)

    zxr1 = _tc1(x_p, W1l, W1r)
    parts1 = _sc_agg_cnt(zxr1, src_p, dst_p).reshape(NC, NPAD, 128)
    cnt1, zxr2 = _tc2(parts1, b1l.reshape(1, D_H), zxr1, W2l, W2r)
    parts2 = _sc_agg(zxr2, src_p, dst_p).reshape(NC, NPAD, 128)
    h2, out = _tc3(parts2, cnt1, b2l.reshape(1, D_H), zxr2, Wlin,
                   blin.reshape(1, 1))
    return (out[:N], h2[:N])
